# baseline (speedup 1.0000x reference)
import functools
import math

import jax
import jax.numpy as jnp
from jax import lax
from jax.experimental import pallas as pl
from jax.experimental.pallas import tpu as pltpu
from jax.experimental.pallas import tpu_sc as plsc

TIME_STEPS = 100000
EMBED_DIM = 128
BATCH = 16384

NUM_CORES = 2
NUM_SUBCORES = 16
NW = NUM_CORES * NUM_SUBCORES
SC_ROWS = 6144
B_PER_W = SC_ROWS // NW

_mesh = plsc.VectorSubcoreMesh(core_axis_name="c", subcore_axis_name="s")


@functools.partial(
    pl.kernel,
    mesh=_mesh,
    out_type=jax.ShapeDtypeStruct((SC_ROWS, EMBED_DIM), jnp.float32),
    scratch_types=[
        pltpu.VMEM((1, B_PER_W), jnp.int32),
        pltpu.VMEM((B_PER_W, EMBED_DIM), jnp.float32),
        pltpu.SemaphoreType.DMA,
    ],
)
def _sc_gather(idx_hbm, table_hbm, out_hbm, idx_v, rows_v, gsem):
    wid = lax.axis_index("s") * NUM_CORES + lax.axis_index("c")
    base = wid * B_PER_W
    pltpu.sync_copy(idx_hbm.at[wid], idx_v)
    pltpu.async_copy(table_hbm.at[idx_v.at[0]], rows_v, gsem).wait()
    pltpu.sync_copy(rows_v, out_hbm.at[pl.ds(base, B_PER_W)])


TC_ROWS = BATCH - SC_ROWS
BLK = 512
NTB = TC_ROWS // BLK


def _tc_body(t_ref, div_ref, out_ref):
    tv = t_ref[0, 0, :].astype(jnp.float32).reshape(BLK, 1)
    ang = tv * div_ref[0, :].reshape(1, EMBED_DIM)
    col = lax.broadcasted_iota(jnp.int32, (BLK, EMBED_DIM), 1)
    phase = jnp.where(col % 2 == 0, 0.0, jnp.float32(math.pi / 2))
    out_ref[...] = jnp.sin(ang + phase)


def _tc_sin(t_tc):
    div = jnp.exp(
        jnp.arange(0, EMBED_DIM, 2, dtype=jnp.float32)
        * -(math.log(10000.0) / EMBED_DIM)
    )
    divfull = jnp.repeat(div, 2).reshape(1, EMBED_DIM)
    t3 = t_tc.reshape(NTB, 1, BLK)
    return pl.pallas_call(
        _tc_body,
        grid=(NTB,),
        in_specs=[
            pl.BlockSpec((1, 1, BLK), lambda i: (i, 0, 0)),
            pl.BlockSpec((1, EMBED_DIM), lambda i: (0, 0)),
        ],
        out_specs=pl.BlockSpec((BLK, EMBED_DIM), lambda i: (i, 0)),
        out_shape=jax.ShapeDtypeStruct((TC_ROWS, EMBED_DIM), jnp.float32),
    )(t3, divfull)


def kernel(t, embeddings):
    ti = t.astype(jnp.int32)
    idx_sc = ti[:SC_ROWS].reshape(NW, 1, B_PER_W)
    out_sc = _sc_gather(idx_sc, embeddings)
    out_tc = _tc_sin(ti[SC_ROWS:])
    out = jnp.concatenate([out_sc, out_tc], axis=0)
    return out[:, :, None, None]


# speedup vs baseline: 1.7382x; 1.7382x over previous
import functools

import jax
import jax.numpy as jnp
from jax import lax
from jax.experimental import pallas as pl
from jax.experimental.pallas import tpu as pltpu
from jax.experimental.pallas import tpu_sc as plsc

TIME_STEPS = 100000
EMBED_DIM = 128
BATCH = 16384

NUM_CORES = 2
NUM_SUBCORES = 16
NW = NUM_CORES * NUM_SUBCORES
B_PER_W = BATCH // NW

_mesh = plsc.VectorSubcoreMesh(core_axis_name="c", subcore_axis_name="s")


@functools.partial(
    pl.kernel,
    mesh=_mesh,
    out_type=jax.ShapeDtypeStruct((BATCH, EMBED_DIM), jnp.float32),
    scratch_types=[
        pltpu.VMEM((1, B_PER_W), jnp.int32),
        pltpu.VMEM((B_PER_W, EMBED_DIM), jnp.float32),
        pltpu.SemaphoreType.DMA,
    ],
)
def _gather_kernel(idx_hbm, table_hbm, out_hbm, idx_v, rows_v, gsem):
    wid = lax.axis_index("s") * NUM_CORES + lax.axis_index("c")
    base = wid * B_PER_W
    pltpu.sync_copy(idx_hbm.at[wid], idx_v)
    pltpu.async_copy(table_hbm.at[idx_v.at[0]], rows_v, gsem).wait()
    pltpu.sync_copy(rows_v, out_hbm.at[pl.ds(base, B_PER_W)])


def kernel(t, embeddings):
    idx = t.astype(jnp.int32).reshape(NW, 1, B_PER_W)
    out = _gather_kernel(idx, embeddings)
    return out[:, :, None, None]
